# baseline (speedup 1.0000x reference)
import functools

import jax
import jax.numpy as jnp
from jax import lax
from jax.experimental import pallas as pl
from jax.experimental.pallas import tpu as pltpu
from jax.experimental.pallas import tpu_sc as plsc

_WORD_DROPOUT = 0.1
_VOCAB = 100000
_D = 128

_NC = 2
_NS = 16
_NW = _NC * _NS
_L = 16

_B = 4096 * 50
_PER_W = _B // _NW
_CHUNK = 64
_NCHUNK = _PER_W // _CHUNK
_NBUF = 4
_BITS_W = 3200

_TC_BLK = 1024
_VPAD = 98 * _TC_BLK


def _tc_scale_body(s_ref, table_ref, out_ref):
    i = pl.program_id(0)
    rows = lax.broadcasted_iota(jnp.int32, (_TC_BLK, _D), 0) + i * _TC_BLK
    out_ref[...] = jnp.where(rows < _VOCAB, table_ref[...] * s_ref[...], 0.0)


def _sc_body(table_hbm, idx_hbm, bits_hbm, out_hbm, idx_v, bits_v,
             rows_v, gsems, ssems):
    wid = lax.axis_index("s") * _NC + lax.axis_index("c")
    base = wid * _PER_W

    pltpu.sync_copy(idx_hbm.at[wid], idx_v)
    pltpu.sync_copy(bits_hbm, bits_v)

    zrow = jnp.full((_L,), _VOCAB, jnp.int32)

    @pl.loop(0, _NCHUNK)
    def _remap(c):
        for p in range(_CHUNK // _L):
            iv = idx_v[c, pl.ds(p * _L, _L)]
            w = plsc.load_gather(bits_v, [lax.shift_right_logical(iv, 5)])
            bit = lax.shift_right_logical(w, iv & 31) & 1
            idx_v[c, pl.ds(p * _L, _L)] = jnp.where(bit == 1, iv, zrow)

    def start_gather(c, slot):
        pltpu.async_copy(table_hbm.at[idx_v.at[c]], rows_v.at[slot],
                         gsems.at[slot])

    def wait_gather(c, slot):
        pltpu.make_async_copy(table_hbm.at[idx_v.at[c]], rows_v.at[slot],
                              gsems.at[slot]).wait()

    def out_slice(c):
        return out_hbm.at[pl.ds(base + c * _CHUNK, _CHUNK)]

    def start_scatter(c, slot):
        pltpu.async_copy(rows_v.at[slot], out_slice(c), ssems.at[slot])

    def drain_scatter(c, slot):
        pltpu.make_async_copy(rows_v.at[slot], out_slice(c),
                              ssems.at[slot]).wait()

    start_gather(0, 0)
    start_gather(1, 1)

    @pl.loop(0, _NCHUNK // _NBUF)
    def _quad(q):
        cb = q * _NBUF
        for j in range(_NBUF):
            c = cb + j
            nslot = (j + 2) % _NBUF

            @pl.when(c >= 2)
            def _():
                drain_scatter(c - 2, nslot)

            @pl.when(c + 2 < _NCHUNK)
            def _():
                start_gather(c + 2, nslot)

            wait_gather(c, j)
            start_scatter(c, j)

    drain_scatter(_NCHUNK - 2, (_NCHUNK - 2) % _NBUF)
    drain_scatter(_NCHUNK - 1, (_NCHUNK - 1) % _NBUF)


def kernel(sentences, embedding_matrix, training):
    p = _WORD_DROPOUT
    keep = jax.random.bernoulli(
        jax.random.key(42), 1.0 - p, (embedding_matrix.shape[0], 1))[:, 0]
    keep_pad = jnp.zeros((_BITS_W * 32,), jnp.uint32).at[:_VOCAB].set(
        keep.astype(jnp.uint32))
    bits = (keep_pad.reshape(_BITS_W, 32)
            << jnp.arange(32, dtype=jnp.uint32)[None, :]).sum(
                axis=1, dtype=jnp.uint32).astype(jnp.int32)
    bits_eff = jnp.where(training, bits, jnp.int32(-1))
    s_full = jnp.full((1, _D), jnp.where(training, 1.0 / (1.0 - p), 1.0),
                      jnp.float32)

    scaled = pl.pallas_call(
        _tc_scale_body,
        grid=(_VPAD // _TC_BLK,),
        in_specs=[
            pl.BlockSpec((1, _D), lambda i: (0, 0)),
            pl.BlockSpec((_TC_BLK, _D), lambda i: (i, 0)),
        ],
        out_specs=pl.BlockSpec((_TC_BLK, _D), lambda i: (i, 0)),
        out_shape=jax.ShapeDtypeStruct((_VPAD, _D), jnp.float32),
    )(s_full, embedding_matrix)

    idx = sentences.reshape(_NW, _NCHUNK, _CHUNK)

    mesh = plsc.VectorSubcoreMesh(core_axis_name="c", subcore_axis_name="s")
    out = pl.kernel(
        _sc_body,
        out_type=jax.ShapeDtypeStruct((_B, _D), jnp.float32),
        mesh=mesh,
        compiler_params=pltpu.CompilerParams(needs_layout_passes=False),
        scratch_types=[
            pltpu.VMEM((_NCHUNK, _CHUNK), jnp.int32),
            pltpu.VMEM((_BITS_W,), jnp.int32),
            pltpu.VMEM((_NBUF, _CHUNK, _D), jnp.float32),
            pltpu.SemaphoreType.DMA((_NBUF,)),
            pltpu.SemaphoreType.DMA((_NBUF,)),
        ],
    )(scaled, idx, bits_eff)
    return out.reshape(sentences.shape[0], sentences.shape[1], _D)


# speedup vs baseline: 3.4295x; 3.4295x over previous
import functools

import jax
import jax.numpy as jnp
from jax import lax
from jax.experimental import pallas as pl
from jax.experimental.pallas import tpu as pltpu
from jax.experimental.pallas import tpu_sc as plsc

_WORD_DROPOUT = 0.1
_VOCAB = 100000
_D = 128

_NC = 2
_NS = 16
_NW = _NC * _NS
_L = 16

_B = 4096 * 50
_PER_W = _B // _NW
_CHUNK = 64
_NCHUNK = _PER_W // _CHUNK
_NBUF = 4
_BITS_W = 3200

_TC_BLK = 1024
_VPAD = 98 * _TC_BLK


def _tc_scale_body(s_ref, table_ref, out_ref):
    i = pl.program_id(0)
    rows = lax.broadcasted_iota(jnp.int32, (_TC_BLK, _D), 0) + i * _TC_BLK
    out_ref[...] = jnp.where(rows < _VOCAB, table_ref[...] * s_ref[...], 0.0)


def _sc_body(table_hbm, idx_hbm, bits_hbm, out_hbm, idx_v, bits_v,
             rows_v, gsems, ssems):
    wid = lax.axis_index("s") * _NC + lax.axis_index("c")
    base = wid * _PER_W

    pltpu.sync_copy(idx_hbm.at[wid], idx_v)
    pltpu.sync_copy(bits_hbm, bits_v)

    @pl.loop(0, _NCHUNK)
    def _remap(c):
        for p in range(_CHUNK // _L):
            iv = idx_v[c, pl.ds(p * _L, _L)]
            w = plsc.load_gather(bits_v, [lax.shift_right_logical(iv, 5)])
            bit = lax.shift_right_logical(w, iv & 31) & 1
            zrow = _VOCAB + (iv & 255)
            idx_v[c, pl.ds(p * _L, _L)] = jnp.where(bit == 1, iv, zrow)

    def start_gather(c, slot):
        pltpu.async_copy(table_hbm.at[idx_v.at[c]], rows_v.at[slot],
                         gsems.at[slot])

    def wait_gather(c, slot):
        pltpu.make_async_copy(table_hbm.at[idx_v.at[c]], rows_v.at[slot],
                              gsems.at[slot]).wait()

    def out_slice(c):
        return out_hbm.at[pl.ds(base + c * _CHUNK, _CHUNK)]

    def start_scatter(c, slot):
        pltpu.async_copy(rows_v.at[slot], out_slice(c), ssems.at[slot])

    def drain_scatter(c, slot):
        pltpu.make_async_copy(rows_v.at[slot], out_slice(c),
                              ssems.at[slot]).wait()

    start_gather(0, 0)
    start_gather(1, 1)

    @pl.loop(0, _NCHUNK // _NBUF)
    def _quad(q):
        cb = q * _NBUF
        for j in range(_NBUF):
            c = cb + j
            nslot = (j + 2) % _NBUF

            @pl.when(c >= 2)
            def _():
                drain_scatter(c - 2, nslot)

            @pl.when(c + 2 < _NCHUNK)
            def _():
                start_gather(c + 2, nslot)

            wait_gather(c, j)
            start_scatter(c, j)

    drain_scatter(_NCHUNK - 2, (_NCHUNK - 2) % _NBUF)
    drain_scatter(_NCHUNK - 1, (_NCHUNK - 1) % _NBUF)


def kernel(sentences, embedding_matrix, training):
    p = _WORD_DROPOUT
    keep = jax.random.bernoulli(
        jax.random.key(42), 1.0 - p, (embedding_matrix.shape[0], 1))[:, 0]
    keep_pad = jnp.zeros((_BITS_W * 32,), jnp.uint32).at[:_VOCAB].set(
        keep.astype(jnp.uint32))
    bits = (keep_pad.reshape(_BITS_W, 32)
            << jnp.arange(32, dtype=jnp.uint32)[None, :]).sum(
                axis=1, dtype=jnp.uint32).astype(jnp.int32)
    bits_eff = jnp.where(training, bits, jnp.int32(-1))
    s_full = jnp.full((1, _D), jnp.where(training, 1.0 / (1.0 - p), 1.0),
                      jnp.float32)

    scaled = pl.pallas_call(
        _tc_scale_body,
        grid=(_VPAD // _TC_BLK,),
        in_specs=[
            pl.BlockSpec((1, _D), lambda i: (0, 0)),
            pl.BlockSpec((_TC_BLK, _D), lambda i: (i, 0)),
        ],
        out_specs=pl.BlockSpec((_TC_BLK, _D), lambda i: (i, 0)),
        out_shape=jax.ShapeDtypeStruct((_VPAD, _D), jnp.float32),
    )(s_full, embedding_matrix)

    idx = sentences.reshape(_NW, _NCHUNK, _CHUNK)

    mesh = plsc.VectorSubcoreMesh(core_axis_name="c", subcore_axis_name="s")
    out = pl.kernel(
        _sc_body,
        out_type=jax.ShapeDtypeStruct((_B, _D), jnp.float32),
        mesh=mesh,
        compiler_params=pltpu.CompilerParams(needs_layout_passes=False),
        scratch_types=[
            pltpu.VMEM((_NCHUNK, _CHUNK), jnp.int32),
            pltpu.VMEM((_BITS_W,), jnp.int32),
            pltpu.VMEM((_NBUF, _CHUNK, _D), jnp.float32),
            pltpu.SemaphoreType.DMA((_NBUF,)),
            pltpu.SemaphoreType.DMA((_NBUF,)),
        ],
    )(scaled, idx, bits_eff)
    return out.reshape(sentences.shape[0], sentences.shape[1], _D)


# speedup vs baseline: 5.5417x; 1.6159x over previous
import jax
import jax.numpy as jnp
from jax import lax
from jax.experimental import pallas as pl
from jax.experimental.pallas import tpu as pltpu
from jax.experimental.pallas import tpu_sc as plsc

_WORD_DROPOUT = 0.1
_VOCAB = 100000
_D = 128

_NC = 2
_NS = 16
_NW = _NC * _NS
_L = 16

_NSENT = 4096
_SLEN = 50
_SPAD = 56
_SENT_PER_W = _NSENT // _NW
_ROWS = 64
_NBUF = 4
_BITS_W = 3200


def _sc_body(table_hbm, idx_hbm, bits_hbm, skeep_hbm, sdrop_hbm, out_hbm,
             idx_v, bits_v, skeep_v, sdrop_v, scales_v, rows_v,
             gsems, ssems):
    wid = lax.axis_index("s") * _NC + lax.axis_index("c")
    sbase = wid * _SENT_PER_W

    pltpu.sync_copy(idx_hbm.at[wid], idx_v)
    pltpu.sync_copy(bits_hbm, bits_v)
    pltpu.sync_copy(skeep_hbm, skeep_v)
    pltpu.sync_copy(sdrop_hbm, sdrop_v)

    def start_gather(r, slot):
        pltpu.async_copy(table_hbm.at[idx_v.at[r]], rows_v.at[slot],
                         gsems.at[slot])

    def wait_gather(r, slot):
        pltpu.make_async_copy(table_hbm.at[idx_v.at[r]], rows_v.at[slot],
                              gsems.at[slot]).wait()

    def out_a(r):
        return out_hbm.at[pl.ds((sbase + 2 * r) * _SPAD, _SPAD)]

    def out_b(r):
        return out_hbm.at[pl.ds((sbase + 2 * r + 1) * _SPAD, _SPAD)]

    def start_scatter(r, slot):
        pltpu.async_copy(rows_v.at[slot].at[pl.ds(0, _SPAD)], out_a(r),
                         ssems.at[slot])
        pltpu.async_copy(rows_v.at[slot].at[pl.ds(64, _SPAD)], out_b(r),
                         ssems.at[slot])

    def drain_scatter(r, slot):
        pltpu.make_async_copy(rows_v.at[slot].at[pl.ds(0, _SPAD)], out_a(r),
                              ssems.at[slot]).wait()
        pltpu.make_async_copy(rows_v.at[slot].at[pl.ds(64, _SPAD)], out_b(r),
                              ssems.at[slot]).wait()

    start_gather(0, 0)
    start_gather(1, 1)

    @pl.loop(0, _ROWS // _NBUF)
    def _quad(q):
        cb = q * _NBUF
        for j in range(_NBUF):
            r = cb + j
            nslot = (j + 2) % _NBUF

            @pl.when(r >= 2)
            def _():
                drain_scatter(r - 2, nslot)

            @pl.when(r + 2 < _ROWS)
            def _():
                start_gather(r + 2, nslot)

            s_keep = skeep_v[...]
            s_drop = sdrop_v[...]
            for p in range(_D // _L):
                iv = idx_v[r, pl.ds(p * _L, _L)]
                w = plsc.load_gather(bits_v, [lax.shift_right_logical(iv, 5)])
                bit = lax.shift_right_logical(w, iv & 31) & 1
                scales_v[pl.ds(p * _L, _L)] = jnp.where(bit == 1, s_keep,
                                                        s_drop)

            wait_gather(r, j)

            @pl.loop(0, _D, unroll=2)
            def _row(rr):
                sc = plsc.load_gather(scales_v,
                                      [jnp.full((_L,), rr, jnp.int32)])
                for p in range(_D // _L):
                    rows_v[j, rr, pl.ds(p * _L, _L)] = (
                        rows_v[j, rr, pl.ds(p * _L, _L)] * sc)

            start_scatter(r, j)

    drain_scatter(_ROWS - 2, (_ROWS - 2) % _NBUF)
    drain_scatter(_ROWS - 1, (_ROWS - 1) % _NBUF)


def kernel(sentences, embedding_matrix, training):
    p = _WORD_DROPOUT
    keep = jax.random.bernoulli(
        jax.random.key(42), 1.0 - p, (embedding_matrix.shape[0], 1))[:, 0]
    keep_pad = jnp.zeros((_BITS_W * 32,), jnp.uint32).at[:_VOCAB].set(
        keep.astype(jnp.uint32))
    bits = (keep_pad.reshape(_BITS_W, 32)
            << jnp.arange(32, dtype=jnp.uint32)[None, :]).sum(
                axis=1, dtype=jnp.uint32).astype(jnp.int32)
    s_drop = jnp.full((_L,), jnp.where(training, 0.0, 1.0), jnp.float32)
    s_keep = jnp.full((_L,), jnp.where(training, 1.0 / (1.0 - p), 1.0),
                      jnp.float32)

    padv = jnp.arange(_NSENT * 14, dtype=jnp.int32) % 256
    idx = jnp.concatenate(
        [sentences.astype(jnp.int32), padv.reshape(_NSENT, 14)], axis=1)
    idx = idx.reshape(_NW, _ROWS, _D)

    mesh = plsc.VectorSubcoreMesh(core_axis_name="c", subcore_axis_name="s")
    out = pl.kernel(
        _sc_body,
        out_type=jax.ShapeDtypeStruct((_NSENT * _SPAD, _D), jnp.float32),
        mesh=mesh,
        compiler_params=pltpu.CompilerParams(needs_layout_passes=False),
        scratch_types=[
            pltpu.VMEM((_ROWS, _D), jnp.int32),
            pltpu.VMEM((_BITS_W,), jnp.int32),
            pltpu.VMEM((_L,), jnp.float32),
            pltpu.VMEM((_L,), jnp.float32),
            pltpu.VMEM((_D,), jnp.float32),
            pltpu.VMEM((_NBUF, _D, _D), jnp.float32),
            pltpu.SemaphoreType.DMA((_NBUF,)),
            pltpu.SemaphoreType.DMA((_NBUF,)),
        ],
    )(embedding_matrix, idx, bits, s_keep, s_drop)
    return out.reshape(_NSENT, _SPAD, _D)[:, :_SLEN, :]


# speedup vs baseline: 5.9180x; 1.0679x over previous
import jax
import jax.numpy as jnp
from jax import lax
from jax.experimental import pallas as pl
from jax.experimental.pallas import tpu as pltpu
from jax.experimental.pallas import tpu_sc as plsc

_WORD_DROPOUT = 0.1
_VOCAB = 100000
_D = 128

_NC = 2
_NS = 16
_NW = _NC * _NS
_L = 16

_NSENT = 4096
_SLEN = 50
_SPAD = 56
_SENT_PER_W = _NSENT // _NW
_CHUNK = 2 * _SPAD
_NCHUNK = _SENT_PER_W // 2
_PER_W = _NCHUNK * _CHUNK
_NBUF = 4
_BITS_W = 3200


def _sc_body(table_hbm, idx_hbm, bits_hbm, skeep_hbm, sdrop_hbm, out_hbm,
             idx_v, bits_v, skeep_v, sdrop_v, scales_v, rows_v,
             gsems, ssems):
    wid = lax.axis_index("s") * _NC + lax.axis_index("c")
    obase = wid * _SENT_PER_W * _SPAD

    pltpu.sync_copy(idx_hbm.at[pl.ds(wid * _PER_W, _PER_W)], idx_v)
    pltpu.sync_copy(bits_hbm, bits_v)
    pltpu.sync_copy(skeep_hbm, skeep_v)
    pltpu.sync_copy(sdrop_hbm, sdrop_v)

    def start_gather(c, slot):
        pltpu.async_copy(table_hbm.at[idx_v.at[pl.ds(c * _CHUNK, _CHUNK)]],
                         rows_v.at[slot], gsems.at[slot])

    def wait_gather(c, slot):
        pltpu.make_async_copy(
            table_hbm.at[idx_v.at[pl.ds(c * _CHUNK, _CHUNK)]],
            rows_v.at[slot], gsems.at[slot]).wait()

    def out_slice(c):
        return out_hbm.at[pl.ds(obase + c * _CHUNK, _CHUNK)]

    def start_scatter(c, slot):
        pltpu.async_copy(rows_v.at[slot], out_slice(c), ssems.at[slot])

    def drain_scatter(c, slot):
        pltpu.make_async_copy(rows_v.at[slot], out_slice(c),
                              ssems.at[slot]).wait()

    start_gather(0, 0)
    start_gather(1, 1)

    @pl.loop(0, _NCHUNK // _NBUF)
    def _quad(q):
        cb = q * _NBUF
        for j in range(_NBUF):
            c = cb + j
            nslot = (j + 2) % _NBUF

            @pl.when(c >= 2)
            def _():
                drain_scatter(c - 2, nslot)

            @pl.when(c + 2 < _NCHUNK)
            def _():
                start_gather(c + 2, nslot)

            s_keep = skeep_v[...]
            s_drop = sdrop_v[...]
            for p in range(_CHUNK // _L):
                iv = idx_v[pl.ds(c * _CHUNK + p * _L, _L)]
                w = plsc.load_gather(bits_v, [lax.shift_right_logical(iv, 5)])
                bit = lax.shift_right_logical(w, iv & 31) & 1
                scales_v[pl.ds(p * _L, _L)] = jnp.where(bit == 1, s_keep,
                                                        s_drop)

            wait_gather(c, j)

            @pl.loop(0, _CHUNK, unroll=4)
            def _row(rr):
                sc = plsc.load_gather(scales_v,
                                      [jnp.full((_L,), rr, jnp.int32)])
                for p in range(_D // _L):
                    rows_v[j, rr, pl.ds(p * _L, _L)] = (
                        rows_v[j, rr, pl.ds(p * _L, _L)] * sc)

            start_scatter(c, j)

    drain_scatter(_NCHUNK - 2, (_NCHUNK - 2) % _NBUF)
    drain_scatter(_NCHUNK - 1, (_NCHUNK - 1) % _NBUF)


def kernel(sentences, embedding_matrix, training):
    p = _WORD_DROPOUT
    keep = jax.random.bernoulli(
        jax.random.key(42), 1.0 - p, (embedding_matrix.shape[0], 1))[:, 0]
    keep_pad = jnp.zeros((_BITS_W * 32,), jnp.uint32).at[:_VOCAB].set(
        keep.astype(jnp.uint32))
    bits = (keep_pad.reshape(_BITS_W, 32)
            << jnp.arange(32, dtype=jnp.uint32)[None, :]).sum(
                axis=1, dtype=jnp.uint32).astype(jnp.int32)
    s_drop = jnp.full((_L,), jnp.where(training, 0.0, 1.0), jnp.float32)
    s_keep = jnp.full((_L,), jnp.where(training, 1.0 / (1.0 - p), 1.0),
                      jnp.float32)

    padv = jnp.arange(_NSENT * (_SPAD - _SLEN), dtype=jnp.int32) % 256
    idx = jnp.concatenate(
        [sentences.astype(jnp.int32),
         padv.reshape(_NSENT, _SPAD - _SLEN)], axis=1).reshape(-1)

    mesh = plsc.VectorSubcoreMesh(core_axis_name="c", subcore_axis_name="s")
    out = pl.kernel(
        _sc_body,
        out_type=jax.ShapeDtypeStruct((_NSENT * _SPAD, _D), jnp.float32),
        mesh=mesh,
        compiler_params=pltpu.CompilerParams(needs_layout_passes=False),
        scratch_types=[
            pltpu.VMEM((_PER_W,), jnp.int32),
            pltpu.VMEM((_BITS_W,), jnp.int32),
            pltpu.VMEM((_L,), jnp.float32),
            pltpu.VMEM((_L,), jnp.float32),
            pltpu.VMEM((_CHUNK,), jnp.float32),
            pltpu.VMEM((_NBUF, _CHUNK, _D), jnp.float32),
            pltpu.SemaphoreType.DMA((_NBUF,)),
            pltpu.SemaphoreType.DMA((_NBUF,)),
        ],
    )(embedding_matrix, idx, bits, s_keep, s_drop)
    return out.reshape(_NSENT, _SPAD, _D)[:, :_SLEN, :]


# speedup vs baseline: 5.9937x; 1.0128x over previous
import jax
import jax.numpy as jnp
from jax import lax
from jax.experimental import pallas as pl
from jax.experimental.pallas import tpu as pltpu
from jax.experimental.pallas import tpu_sc as plsc

_WORD_DROPOUT = 0.1
_VOCAB = 100000
_D = 128

_NC = 2
_NS = 16
_NW = _NC * _NS
_L = 16

_NSENT = 4096
_SLEN = 50
_SPAD = 56
_SENT_PER_W = _NSENT // _NW
_CHUNK = 2 * _SPAD
_NCHUNK = _SENT_PER_W // 2
_PER_W = _NCHUNK * _CHUNK
_NBUF = 8
_BITS_W = 3200


def _sc_body(table_hbm, idx_hbm, bits_hbm, skeep_hbm, sdrop_hbm, out_hbm,
             idx_v, bits_v, skeep_v, sdrop_v, scales_v, rows_v,
             gsems, ssems):
    wid = lax.axis_index("s") * _NC + lax.axis_index("c")
    obase = wid * _SENT_PER_W * _SPAD

    pltpu.sync_copy(idx_hbm.at[pl.ds(wid * _PER_W, _PER_W)], idx_v)
    pltpu.sync_copy(bits_hbm, bits_v)
    pltpu.sync_copy(skeep_hbm, skeep_v)
    pltpu.sync_copy(sdrop_hbm, sdrop_v)

    def start_gather(c, slot):
        pltpu.async_copy(table_hbm.at[idx_v.at[pl.ds(c * _CHUNK, _CHUNK)]],
                         rows_v.at[slot], gsems.at[slot])

    def wait_gather(c, slot):
        pltpu.make_async_copy(
            table_hbm.at[idx_v.at[pl.ds(c * _CHUNK, _CHUNK)]],
            rows_v.at[slot], gsems.at[slot]).wait()

    def out_slice(c):
        return out_hbm.at[pl.ds(obase + c * _CHUNK, _CHUNK)]

    def start_scatter(c, slot):
        pltpu.async_copy(rows_v.at[slot], out_slice(c), ssems.at[slot])

    def drain_scatter(c, slot):
        pltpu.make_async_copy(rows_v.at[slot], out_slice(c),
                              ssems.at[slot]).wait()

    start_gather(0, 0)
    start_gather(1, 1)
    start_gather(2, 2)
    start_gather(3, 3)

    @pl.loop(0, _NCHUNK // _NBUF)
    def _quad(q):
        cb = q * _NBUF
        for j in range(_NBUF):
            c = cb + j
            nslot = (j + 4) % _NBUF

            @pl.when(c >= 4)
            def _():
                drain_scatter(c - 4, nslot)

            @pl.when(c + 4 < _NCHUNK)
            def _():
                start_gather(c + 4, nslot)

            s_keep = skeep_v[...]
            s_drop = sdrop_v[...]
            for p in range(_CHUNK // _L):
                iv = idx_v[pl.ds(c * _CHUNK + p * _L, _L)]
                w = plsc.load_gather(bits_v, [lax.shift_right_logical(iv, 5)])
                bit = lax.shift_right_logical(w, iv & 31) & 1
                scales_v[pl.ds(p * _L, _L)] = jnp.where(bit == 1, s_keep,
                                                        s_drop)

            wait_gather(c, j)

            @pl.loop(0, _CHUNK, unroll=4)
            def _row(rr):
                sc = plsc.load_gather(scales_v,
                                      [jnp.full((_L,), rr, jnp.int32)])
                for p in range(_D // _L):
                    rows_v[j, rr, pl.ds(p * _L, _L)] = (
                        rows_v[j, rr, pl.ds(p * _L, _L)] * sc)

            start_scatter(c, j)

    for t in range(4):
        drain_scatter(_NCHUNK - 4 + t, (_NCHUNK - 4 + t) % _NBUF)


def kernel(sentences, embedding_matrix, training):
    p = _WORD_DROPOUT
    keep = jax.random.bernoulli(
        jax.random.key(42), 1.0 - p, (embedding_matrix.shape[0], 1))[:, 0]
    keep_pad = jnp.zeros((_BITS_W * 32,), jnp.uint32).at[:_VOCAB].set(
        keep.astype(jnp.uint32))
    bits = (keep_pad.reshape(_BITS_W, 32)
            << jnp.arange(32, dtype=jnp.uint32)[None, :]).sum(
                axis=1, dtype=jnp.uint32).astype(jnp.int32)
    s_drop = jnp.full((_L,), jnp.where(training, 0.0, 1.0), jnp.float32)
    s_keep = jnp.full((_L,), jnp.where(training, 1.0 / (1.0 - p), 1.0),
                      jnp.float32)

    padv = jnp.arange(_NSENT * (_SPAD - _SLEN), dtype=jnp.int32) % 256
    idx = jnp.concatenate(
        [sentences.astype(jnp.int32),
         padv.reshape(_NSENT, _SPAD - _SLEN)], axis=1).reshape(-1)

    mesh = plsc.VectorSubcoreMesh(core_axis_name="c", subcore_axis_name="s")
    out = pl.kernel(
        _sc_body,
        out_type=jax.ShapeDtypeStruct((_NSENT * _SPAD, _D), jnp.float32),
        mesh=mesh,
        compiler_params=pltpu.CompilerParams(needs_layout_passes=False),
        scratch_types=[
            pltpu.VMEM((_PER_W,), jnp.int32),
            pltpu.VMEM((_BITS_W,), jnp.int32),
            pltpu.VMEM((_L,), jnp.float32),
            pltpu.VMEM((_L,), jnp.float32),
            pltpu.VMEM((_CHUNK,), jnp.float32),
            pltpu.VMEM((_NBUF, _CHUNK, _D), jnp.float32),
            pltpu.SemaphoreType.DMA((_NBUF,)),
            pltpu.SemaphoreType.DMA((_NBUF,)),
        ],
    )(embedding_matrix, idx, bits, s_keep, s_drop)
    return out.reshape(_NSENT, _SPAD, _D)[:, :_SLEN, :]


# speedup vs baseline: 6.8159x; 1.1372x over previous
import jax
import jax.numpy as jnp
from jax import lax
from jax.experimental import pallas as pl
from jax.experimental.pallas import tpu as pltpu
from jax.experimental.pallas import tpu_sc as plsc

_WORD_DROPOUT = 0.1
_VOCAB = 100000
_D = 128

_NC = 2
_NS = 16
_NW = _NC * _NS
_L = 16

_NSENT = 4096
_SLEN = 50
_SPAD = 56
_SENT_PER_W = _NSENT // _NW
_CHUNK = 2 * _SPAD
_NCHUNK = _SENT_PER_W // 2
_PER_W = _NCHUNK * _CHUNK
_NBUF = 8
_BITS_W = 3200


def _sc_body(table_hbm, idx_hbm, bits_hbm, skeep_hbm, sdrop_hbm, out_hbm,
             idx_v, bits_v, skeep_v, sdrop_v, scales_v, rows_v,
             gsems, ssems):
    wid = lax.axis_index("s") * _NC + lax.axis_index("c")
    sbase = wid * _SENT_PER_W

    pltpu.sync_copy(idx_hbm.at[pl.ds(wid * _PER_W, _PER_W)], idx_v)
    pltpu.sync_copy(bits_hbm, bits_v)
    pltpu.sync_copy(skeep_hbm, skeep_v)
    pltpu.sync_copy(sdrop_hbm, sdrop_v)

    def start_gather(c, slot):
        pltpu.async_copy(table_hbm.at[idx_v.at[pl.ds(c * _CHUNK, _CHUNK)]],
                         rows_v.at[slot], gsems.at[slot])

    def wait_gather(c, slot):
        pltpu.make_async_copy(
            table_hbm.at[idx_v.at[pl.ds(c * _CHUNK, _CHUNK)]],
            rows_v.at[slot], gsems.at[slot]).wait()

    def start_scatter(c, slot):
        pltpu.async_copy(rows_v.at[slot].at[pl.ds(0, _SLEN)],
                         out_hbm.at[sbase + 2 * c], ssems.at[slot])
        pltpu.async_copy(rows_v.at[slot].at[pl.ds(_SPAD, _SLEN)],
                         out_hbm.at[sbase + 2 * c + 1], ssems.at[slot])

    def drain_scatter(c, slot):
        pltpu.make_async_copy(rows_v.at[slot].at[pl.ds(0, _SLEN)],
                              out_hbm.at[sbase + 2 * c],
                              ssems.at[slot]).wait()
        pltpu.make_async_copy(rows_v.at[slot].at[pl.ds(_SPAD, _SLEN)],
                              out_hbm.at[sbase + 2 * c + 1],
                              ssems.at[slot]).wait()

    start_gather(0, 0)
    start_gather(1, 1)
    start_gather(2, 2)
    start_gather(3, 3)

    @pl.loop(0, _NCHUNK // _NBUF)
    def _quad(q):
        cb = q * _NBUF
        for j in range(_NBUF):
            c = cb + j
            nslot = (j + 4) % _NBUF

            @pl.when(c >= 4)
            def _():
                drain_scatter(c - 4, nslot)

            @pl.when(c + 4 < _NCHUNK)
            def _():
                start_gather(c + 4, nslot)

            s_keep = skeep_v[...]
            s_drop = sdrop_v[...]
            for p in range(_CHUNK // _L):
                iv = idx_v[pl.ds(c * _CHUNK + p * _L, _L)]
                w = plsc.load_gather(bits_v, [lax.shift_right_logical(iv, 5)])
                bit = lax.shift_right_logical(w, iv & 31) & 1
                scales_v[pl.ds(p * _L, _L)] = jnp.where(bit == 1, s_keep,
                                                        s_drop)

            wait_gather(c, j)

            @pl.loop(0, _CHUNK, unroll=4)
            def _row(rr):
                sc = plsc.load_gather(scales_v,
                                      [jnp.full((_L,), rr, jnp.int32)])
                for p in range(_D // _L):
                    rows_v[j, rr, pl.ds(p * _L, _L)] = (
                        rows_v[j, rr, pl.ds(p * _L, _L)] * sc)

            start_scatter(c, j)

    for t in range(4):
        drain_scatter(_NCHUNK - 4 + t, (_NCHUNK - 4 + t) % _NBUF)


def kernel(sentences, embedding_matrix, training):
    p = _WORD_DROPOUT
    keep = jax.random.bernoulli(
        jax.random.key(42), 1.0 - p, (embedding_matrix.shape[0], 1))[:, 0]
    keep_pad = jnp.zeros((_BITS_W * 32,), jnp.uint32).at[:_VOCAB].set(
        keep.astype(jnp.uint32))
    bits = (keep_pad.reshape(_BITS_W, 32)
            << jnp.arange(32, dtype=jnp.uint32)[None, :]).sum(
                axis=1, dtype=jnp.uint32).astype(jnp.int32)
    s_drop = jnp.full((_L,), jnp.where(training, 0.0, 1.0), jnp.float32)
    s_keep = jnp.full((_L,), jnp.where(training, 1.0 / (1.0 - p), 1.0),
                      jnp.float32)

    padv = jnp.arange(_NSENT * (_SPAD - _SLEN), dtype=jnp.int32) % 256
    idx = jnp.concatenate(
        [sentences.astype(jnp.int32),
         padv.reshape(_NSENT, _SPAD - _SLEN)], axis=1).reshape(-1)

    mesh = plsc.VectorSubcoreMesh(core_axis_name="c", subcore_axis_name="s")
    out = pl.kernel(
        _sc_body,
        out_type=jax.ShapeDtypeStruct((_NSENT, _SLEN, _D), jnp.float32),
        mesh=mesh,
        compiler_params=pltpu.CompilerParams(needs_layout_passes=False),
        scratch_types=[
            pltpu.VMEM((_PER_W,), jnp.int32),
            pltpu.VMEM((_BITS_W,), jnp.int32),
            pltpu.VMEM((_L,), jnp.float32),
            pltpu.VMEM((_L,), jnp.float32),
            pltpu.VMEM((_CHUNK,), jnp.float32),
            pltpu.VMEM((_NBUF, _CHUNK, _D), jnp.float32),
            pltpu.SemaphoreType.DMA((_NBUF,)),
            pltpu.SemaphoreType.DMA((_NBUF,)),
        ],
    )(embedding_matrix, idx, bits, s_keep, s_drop)
    return out
